# Initial kernel scaffold; baseline (speedup 1.0000x reference)
import functools

import jax
import jax.numpy as jnp
from jax import lax
from jax.experimental import pallas as pl
from jax.experimental.pallas import tpu as pltpu
from jax.experimental.pallas import tpu_sc as plsc

NC = 2
NS = 16
NW = NC * NS

CHUNK = 128
NBUF = 8


@functools.partial(jax.jit, static_argnums=(2, 3))
def _gather(x_flat, weight, b_per_w, d):
    nchunk = b_per_w // CHUNK
    ngroup = nchunk // NBUF
    mesh = plsc.VectorSubcoreMesh(core_axis_name="c", subcore_axis_name="s")

    @functools.partial(
        pl.kernel,
        out_type=jax.ShapeDtypeStruct((NW * b_per_w, d), jnp.float32),
        mesh=mesh,
        scratch_types=(
            [pltpu.VMEM((b_per_w,), jnp.int32)]
            + [pltpu.VMEM((CHUNK, d), jnp.float32) for _ in range(NBUF)]
            + [pltpu.SemaphoreType.DMA for _ in range(NBUF)]
        ),
    )
    def k(x_hbm, w_hbm, out_hbm, idx_v, *bufs_and_sems):
        bufs = bufs_and_sems[:NBUF]
        sems = bufs_and_sems[NBUF:]
        wid = lax.axis_index("s") * NC + lax.axis_index("c")
        base = wid * b_per_w
        pltpu.sync_copy(x_hbm.at[pl.ds(base, b_per_w)], idx_v)

        def fire(b, chunk):
            pltpu.async_copy(
                w_hbm.at[idx_v.at[pl.ds(chunk * CHUNK, CHUNK)]], bufs[b], sems[b]
            )

        for b in range(NBUF):
            fire(b, b)

        @pl.loop(0, ngroup)
        def _(g):
            for b in range(NBUF):
                chunk = g * NBUF + b
                pltpu.make_async_copy(
                    w_hbm.at[pl.ds(0, CHUNK)], bufs[b], sems[b]
                ).wait()
                pltpu.sync_copy(
                    bufs[b], out_hbm.at[pl.ds(base + chunk * CHUNK, CHUNK)]
                )
                nxt = chunk + NBUF

                @pl.when(nxt < nchunk)
                def _():
                    fire(b, nxt)

    return k(x_flat, weight)


def kernel(x, weight):
    n, s = x.shape
    d = weight.shape[1]
    b = n * s
    x_flat = x.reshape(b).astype(jnp.int32)
    out = _gather(x_flat, weight, b // NW, d)
    return out.reshape(n, s, d)


# speedup vs baseline: 1.8749x; 1.8749x over previous
import functools

import jax
import jax.numpy as jnp
from jax import lax
from jax.experimental import pallas as pl
from jax.experimental.pallas import tpu as pltpu
from jax.experimental.pallas import tpu_sc as plsc

NC = 2
NS = 16
NW = NC * NS

CHUNK = 128
NBUF = 8


@functools.partial(jax.jit, static_argnums=(2, 3))
def _gather(x_flat, weight, b_per_w, d):
    nchunk = b_per_w // CHUNK
    ngroup = nchunk // NBUF
    mesh = plsc.VectorSubcoreMesh(core_axis_name="c", subcore_axis_name="s")

    @functools.partial(
        pl.kernel,
        out_type=jax.ShapeDtypeStruct((NW * b_per_w, d), jnp.float32),
        mesh=mesh,
        scratch_types=(
            [pltpu.VMEM((b_per_w,), jnp.int32)]
            + [pltpu.VMEM((CHUNK, d), jnp.float32) for _ in range(NBUF)]
            + [pltpu.SemaphoreType.DMA for _ in range(NBUF)]
        ),
        compiler_params=pltpu.CompilerParams(use_tc_tiling_on_sc=False),
    )
    def k(x_hbm, w_hbm, out_hbm, idx_v, *bufs_and_sems):
        bufs = bufs_and_sems[:NBUF]
        sems = bufs_and_sems[NBUF:]
        wid = lax.axis_index("s") * NC + lax.axis_index("c")
        base = wid * b_per_w
        pltpu.sync_copy(x_hbm.at[pl.ds(base, b_per_w)], idx_v)

        def fire(b, chunk):
            pltpu.async_copy(
                w_hbm.at[idx_v.at[pl.ds(chunk * CHUNK, CHUNK)]], bufs[b], sems[b]
            )

        for b in range(NBUF):
            fire(b, b)

        @pl.loop(0, ngroup)
        def _(g):
            for b in range(NBUF):
                chunk = g * NBUF + b
                pltpu.make_async_copy(
                    w_hbm.at[pl.ds(0, CHUNK)], bufs[b], sems[b]
                ).wait()
                pltpu.sync_copy(
                    bufs[b], out_hbm.at[pl.ds(base + chunk * CHUNK, CHUNK)]
                )
                nxt = chunk + NBUF

                @pl.when(nxt < nchunk)
                def _():
                    fire(b, nxt)

    return k(x_flat, weight)


def kernel(x, weight):
    n, s = x.shape
    d = weight.shape[1]
    b = n * s
    x_flat = x.reshape(b).astype(jnp.int32)
    out = _gather(x_flat, weight, b // NW, d)
    return out.reshape(n, s, d)
